# Initial kernel scaffold; baseline (speedup 1.0000x reference)
import jax
import jax.numpy as jnp
from jax.experimental import pallas as pl


def kernel(x, edge_index, edge_weight, W_enc, b_enc, ln_gamma, ln_beta, W_l, b_l, W_r, b_r, W_e, att, bias_out, ln_f_gamma, ln_f_beta, W_fc, b_fc):
    raise NotImplementedError("write your pallas kernel here")


# speedup vs baseline: 12.4875x; 12.4875x over previous
import functools

import jax
import jax.numpy as jnp
from jax import lax
from jax.experimental import pallas as pl
from jax.experimental.pallas import tpu as pltpu
from jax.experimental.pallas import tpu_sc as plsc

N = 10000
E = 320000
H = 128
L = 3

NC = 2
NS = 16
NW = NC * NS
EPW = E // NW
CH = 80
NCH = EPW // CH
NPAD = 10240
RPT = NPAD // NS

_mesh = plsc.VectorSubcoreMesh(core_axis_name="c", subcore_axis_name="s")


ECH = E // CH


@functools.partial(
    pl.kernel,
    mesh=_mesh,
    compiler_params=pltpu.CompilerParams(needs_layout_passes=False,
                                         use_tc_tiling_on_sc=False),
    out_type=(
        jax.ShapeDtypeStruct((NC * NPAD, H), jnp.float32),
        jax.ShapeDtypeStruct((NC * NPAD,), jnp.float32),
    ),
    scratch_types=[
        pltpu.VMEM((CH,), jnp.int32),
        pltpu.VMEM((CH,), jnp.int32),
        pltpu.VMEM((CH,), jnp.int32),
        pltpu.VMEM((CH,), jnp.int32),
        pltpu.VMEM((CH,), jnp.int32),
        pltpu.VMEM((CH,), jnp.int32),
        pltpu.VMEM((CH,), jnp.float32),
        pltpu.VMEM((CH,), jnp.float32),
        pltpu.VMEM((CH,), jnp.float32),
        pltpu.VMEM((CH,), jnp.float32),
        pltpu.VMEM((CH, H), jnp.float32),
        pltpu.VMEM((CH, H), jnp.float32),
        pltpu.VMEM((CH, H), jnp.float32),
        pltpu.VMEM((CH, H), jnp.float32),
        pltpu.VMEM((256,), jnp.float32),
        pltpu.VMEM((H,), jnp.float32),
        pltpu.VMEM((H,), jnp.float32),
        pltpu.VMEM((RPT,), jnp.float32),
        pltpu.VMEM_SHARED((NPAD,), jnp.float32),
        pltpu.VMEM_SHARED((NPAD, H), jnp.float32),
        pltpu.SemaphoreType.DMA,
        pltpu.SemaphoreType.DMA,
        pltpu.SemaphoreType.DMA,
        pltpu.SemaphoreType.DMA,
        pltpu.SemaphoreType.DMA,
        pltpu.SemaphoreType.DMA,
        pltpu.SemaphoreType.DMA,
        pltpu.SemaphoreType.DMA,
        pltpu.SemaphoreType.DMA,
        pltpu.SemaphoreType.DMA,
    ],
)
def _edge_fused(xl_hbm, xr_hbm, src2_hbm, dst2_hbm, ew2_hbm, att_hbm, we_hbm,
                z_hbm,
                outp_hbm, denom_hbm,
                src0_v, src1_v, dst0_v, dst1_v, sdst0_v, sdst1_v,
                ew0_v, ew1_v, ae0_v, ae1_v,
                xlr0_v, xrr0_v, xlr1_v, xrr1_v,
                st_v, att_v, we_v, z_v, denom_sp, acc_sp,
                semI0, semI1, semA0, semB0, semA1, semB1,
                semD0, semD1, semS0, semS1):
    cid = lax.axis_index("c")
    sid = lax.axis_index("s")
    wid = sid * NC + cid
    rbase = wid * NCH

    zero16 = jnp.zeros((16,), jnp.float32)

    def _zb(i, _):
        z_v[pl.ds(i * 16, 16)] = zero16
        return 0

    lax.fori_loop(0, RPT // 16, _zb, 0)
    pltpu.sync_copy(z_v, denom_sp.at[pl.ds(sid * RPT, RPT)])
    pltpu.sync_copy(z_hbm, acc_sp.at[pl.ds(sid * RPT, RPT)])

    pltpu.sync_copy(att_hbm, att_v)
    pltpu.sync_copy(we_hbm, we_v)
    plsc.subcore_barrier()

    attr = [att_v[pl.ds(16 * k, 16)] for k in range(H // 16)]
    wer = [we_v[pl.ds(16 * k, 16)] for k in range(H // 16)]
    bufs = ((src0_v, dst0_v, sdst0_v, ew0_v, ae0_v, xlr0_v, xrr0_v,
             semI0, semA0, semB0, semD0, semS0),
            (src1_v, dst1_v, sdst1_v, ew1_v, ae1_v, xlr1_v, xrr1_v,
             semI1, semA1, semB1, semD1, semS1))

    def _fire_idx(c, b):
        src_b, dst_b, sdst_b, ew_b, ae_b = bufs[b][:5]
        semI = bufs[b][7]
        pltpu.async_copy(src2_hbm.at[rbase + c], src_b, semI)
        pltpu.async_copy(dst2_hbm.at[rbase + c], dst_b, semI)
        pltpu.async_copy(ew2_hbm.at[rbase + c], ew_b, semI)

    def _wait_idx(b):
        src_b, dst_b, sdst_b, ew_b, ae_b = bufs[b][:5]
        semI = bufs[b][7]
        pltpu.make_async_copy(src2_hbm.at[rbase], src_b, semI).wait()
        pltpu.make_async_copy(dst2_hbm.at[rbase], dst_b, semI).wait()
        pltpu.make_async_copy(ew2_hbm.at[rbase], ew_b, semI).wait()

    def _fire_rows(b, wait_scatter):
        (src_b, dst_b, sdst_b, ew_b, ae_b, xlr_b, xrr_b,
         semI, semA, semB, semD, semS) = bufs[b]
        _wait_idx(b)
        if wait_scatter:
            pltpu.make_async_copy(z_hbm.at[pl.ds(0, CH)], xlr_b, semS).wait()
            pltpu.make_async_copy(ew2_hbm.at[rbase], ae_b, semD).wait()
        pltpu.async_copy(xl_hbm.at[src_b], xlr_b, semA)
        pltpu.async_copy(xr_hbm.at[dst_b], xrr_b, semB)

    def _compute(c, b):
        (src_b, dst_b, sdst_b, ew_b, ae_b, xlr_b, xrr_b,
         semI, semA, semB, semD, semS) = bufs[b]
        pltpu.make_async_copy(xl_hbm.at[src_b], xlr_b, semA).wait()
        pltpu.make_async_copy(xr_hbm.at[dst_b], xrr_b, semB).wait()

        for g in range(CH // 16):
            s = pl.ds(16 * g, 16)
            sdst_b[s] = dst_b[s]

        def grp_body(g, _):
            s = pl.ds(16 * g, 16)
            for l in range(16):
                e = 16 * g + l
                ewes = plsc.load_gather(ew_b, [jnp.full((16,), e, jnp.int32)])
                acc = jnp.zeros((16,), jnp.float32)
                for k in range(H // 16):
                    sk = pl.ds(16 * k, 16)
                    v = xlr_b[e, sk] + xrr_b[e, sk] + ewes * wer[k]
                    acc = acc + attr[k] * jnp.maximum(v, 0.2 * v)
                st_v[pl.ds(16 * l, 16)] = acc
            base16 = lax.iota(jnp.int32, 16) * 16
            tot = jnp.zeros((16,), jnp.float32)
            for i in range(16):
                tot = tot + plsc.load_gather(st_v, [base16 + i])
            ae_b[s] = jnp.exp(tot)
            for l in range(16):
                e = 16 * g + l
                asp = plsc.load_gather(ae_b, [jnp.full((16,), e, jnp.int32)])
                for k in range(H // 16):
                    sk = pl.ds(16 * k, 16)
                    xlr_b[e, sk] = xlr_b[e, sk] * asp
            return 0

        lax.fori_loop(0, CH // 16, grp_body, 0)
        pltpu.async_copy(ae_b, denom_sp.at[sdst_b], semD, add=True)
        pltpu.async_copy(xlr_b, acc_sp.at[sdst_b], semS, add=True)
        _fire_idx(jnp.minimum(c + 2, NCH - 1), b)

    _fire_idx(0, 0)
    _fire_idx(1, 1)
    _fire_rows(0, False)
    _fire_rows(1, False)
    _compute(0, 0)
    _fire_rows(0, True)
    _compute(1, 1)

    def pipe_body(t, _):
        c0 = 2 * t
        _fire_rows(1, True)
        _compute(c0, 0)
        _fire_rows(0, True)
        _compute(c0 + 1, 1)
        return 0

    lax.fori_loop(1, (NCH - 1) // 2, pipe_body, 0)
    _compute(NCH - 1, 0)

    pltpu.make_async_copy(z_hbm.at[pl.ds(0, CH)], xlr1_v, semS1).wait()
    pltpu.make_async_copy(z_hbm.at[pl.ds(0, CH)], xlr0_v, semS0).wait()
    pltpu.make_async_copy(ew2_hbm.at[rbase], ae1_v, semD1).wait()
    pltpu.make_async_copy(ew2_hbm.at[rbase], ae0_v, semD0).wait()
    _wait_idx(1)
    _wait_idx(0)

    plsc.subcore_barrier()
    pltpu.sync_copy(denom_sp.at[pl.ds(sid * RPT, RPT)],
                    denom_hbm.at[pl.ds(cid * NPAD + sid * RPT, RPT)])
    pltpu.sync_copy(acc_sp.at[pl.ds(sid * RPT, RPT)],
                    outp_hbm.at[pl.ds(cid * NPAD + sid * RPT, RPT)])


BR = 400


def _enc_body(x_ref, w_ref, b_ref, o_ref):
    o_ref[:] = jnp.dot(x_ref[:], w_ref[:],
                       preferred_element_type=jnp.float32) + b_ref[:]


def _encode(x, W_enc, b_enc):
    return pl.pallas_call(
        _enc_body,
        grid=(N // BR,),
        in_specs=[
            pl.BlockSpec((BR, H), lambda i: (i, 0)),
            pl.BlockSpec((H, H), lambda i: (0, 0)),
            pl.BlockSpec((1, H), lambda i: (0, 0)),
        ],
        out_specs=pl.BlockSpec((BR, H), lambda i: (i, 0)),
        out_shape=jax.ShapeDtypeStruct((N, H), jnp.float32),
    )(x, W_enc, b_enc.reshape(1, H))


def _ln_relu(h, g, b):
    mu = jnp.mean(h, axis=1, keepdims=True)
    var = jnp.mean((h - mu) ** 2, axis=1, keepdims=True)
    return jnp.maximum(g * (h - mu) * lax.rsqrt(var + 1e-5) + b, 0.0)


def _residual(h, r0, r1, d0, d1, badd):
    return h + (r0 + r1) / (d0 + d1 + 1e-16) + badd


def _dense_body(h_ref, r0_ref, r1_ref, d0_ref, d1_ref, badd_ref, g_ref, b_ref,
                wl_ref, bl_ref, wr_ref, br_ref,
                hn_ref, xl_ref, xr_ref):
    hv = _residual(h_ref[:], r0_ref[:], r1_ref[:], d0_ref[:], d1_ref[:],
                   badd_ref[:])
    hn_ref[:] = hv
    a = _ln_relu(hv, g_ref[:], b_ref[:])
    xl_ref[:] = jnp.dot(a, wl_ref[:], preferred_element_type=jnp.float32) + bl_ref[:]
    xr_ref[:] = jnp.dot(a, wr_ref[:], preferred_element_type=jnp.float32) + br_ref[:]


def _dense_layer(h, r0, r1, d0, d1, badd, g, b, Wl, bl, Wr, br):
    vec = pl.BlockSpec((1, H), lambda i: (0, 0))
    mat = pl.BlockSpec((H, H), lambda i: (0, 0))
    row = pl.BlockSpec((BR, H), lambda i: (i, 0))
    col = pl.BlockSpec((BR, 1), lambda i: (i, 0))
    return pl.pallas_call(
        _dense_body,
        grid=(N // BR,),
        in_specs=[row, row, row, col, col, vec, vec, vec, mat, vec, mat, vec],
        out_specs=(row, row, row),
        out_shape=(jax.ShapeDtypeStruct((N, H), jnp.float32),
                   jax.ShapeDtypeStruct((N, H), jnp.float32),
                   jax.ShapeDtypeStruct((N, H), jnp.float32)),
    )(h, r0, r1, d0, d1, badd.reshape(1, H), g.reshape(1, H), b.reshape(1, H),
      Wl, bl.reshape(1, H), Wr, br.reshape(1, H))


def _final_body(h_ref, r0_ref, r1_ref, d0_ref, d1_ref, badd_ref, g_ref, b_ref,
                wf_ref, bf_ref, y_ref):
    hv = _residual(h_ref[:], r0_ref[:], r1_ref[:], d0_ref[:], d1_ref[:],
                   badd_ref[:])
    a = _ln_relu(hv, g_ref[:], b_ref[:])
    y_ref[:] = jnp.dot(a, wf_ref[:], preferred_element_type=jnp.float32) + bf_ref[:]


def _final_layer(h, r0, r1, d0, d1, badd, g, b, W_fc, b_fc):
    vec = pl.BlockSpec((1, H), lambda i: (0, 0))
    row = pl.BlockSpec((BR, H), lambda i: (i, 0))
    col = pl.BlockSpec((BR, 1), lambda i: (i, 0))
    return pl.pallas_call(
        _final_body,
        grid=(N // BR,),
        in_specs=[row, row, row, col, col, vec, vec, vec,
                  pl.BlockSpec((H, 1), lambda i: (0, 0)),
                  pl.BlockSpec((1, 1), lambda i: (0, 0))],
        out_specs=pl.BlockSpec((BR, 1), lambda i: (i, 0)),
        out_shape=jax.ShapeDtypeStruct((N, 1), jnp.float32),
    )(h, r0, r1, d0, d1, badd.reshape(1, H), g.reshape(1, H), b.reshape(1, H),
      W_fc, b_fc.reshape(1, 1))


def kernel(x, edge_index, edge_weight, W_enc, b_enc, ln_gamma, ln_beta,
           W_l, b_l, W_r, b_r, W_e, att, bias_out, ln_f_gamma, ln_f_beta,
           W_fc, b_fc):
    srcf = edge_index[0].astype(jnp.int32)
    dstf = edge_index[1].astype(jnp.int32)
    src2 = srcf.reshape(ECH, CH)
    dst2 = dstf.reshape(ECH, CH)
    ew2 = edge_weight[:, 0].reshape(ECH, CH)
    zrow = jnp.zeros((RPT, H), jnp.float32)
    zres = jnp.zeros((N, H), jnp.float32)

    h = _encode(x, W_enc, b_enc)
    r0 = zres
    r1 = zres
    d0 = jnp.ones((N, 1), jnp.float32)
    d1 = jnp.zeros((N, 1), jnp.float32)
    badd = jnp.zeros((H,), jnp.float32)
    for i in range(L):
        h, xl, xr = _dense_layer(h, r0, r1, d0, d1, badd,
                                 ln_gamma[i], ln_beta[i],
                                 W_l[i], b_l[i], W_r[i], b_r[i])
        outp, denom = _edge_fused(xl, xr, src2, dst2, ew2, att[i], W_e[i][0],
                                  zrow)
        r0 = outp[:N]
        r1 = outp[NPAD:NPAD + N]
        d0 = denom[:N].reshape(N, 1)
        d1 = denom[NPAD:NPAD + N].reshape(N, 1)
        badd = bias_out[i]
    return _final_layer(h, r0, r1, d0, d1, badd, ln_f_gamma, ln_f_beta,
                        W_fc, b_fc)


# speedup vs baseline: 13.5191x; 1.0826x over previous
import functools

import jax
import jax.numpy as jnp
from jax import lax
from jax.experimental import pallas as pl
from jax.experimental.pallas import tpu as pltpu
from jax.experimental.pallas import tpu_sc as plsc

N = 10000
E = 320000
H = 128
L = 3

NC = 2
NS = 16
NW = NC * NS
EPW = E // NW
CH = 80
NCH = EPW // CH
NPAD = 10240
RPT = NPAD // NS

_mesh = plsc.VectorSubcoreMesh(core_axis_name="c", subcore_axis_name="s")


ECH = E // CH


@functools.partial(
    pl.kernel,
    mesh=_mesh,
    compiler_params=pltpu.CompilerParams(needs_layout_passes=False,
                                        use_tc_tiling_on_sc=False),
    out_type=(
        jax.ShapeDtypeStruct((ECH, CH), jnp.float32),
        jax.ShapeDtypeStruct((NC * NPAD,), jnp.float32),
    ),
    scratch_types=[
        pltpu.VMEM((NCH, CH), jnp.int32),
        pltpu.VMEM((NCH, CH), jnp.int32),
        pltpu.VMEM((NCH, CH), jnp.float32),
        pltpu.VMEM((NCH, CH), jnp.float32),
        pltpu.VMEM((CH,), jnp.float32),
        pltpu.VMEM((CH, H), jnp.float32),
        pltpu.VMEM((CH, H), jnp.float32),
        pltpu.VMEM((CH, H), jnp.float32),
        pltpu.VMEM((CH, H), jnp.float32),
        pltpu.VMEM((256,), jnp.float32),
        pltpu.VMEM((H,), jnp.float32),
        pltpu.VMEM((H,), jnp.float32),
        pltpu.VMEM((RPT,), jnp.float32),
        pltpu.VMEM_SHARED((NPAD,), jnp.float32),
        pltpu.SemaphoreType.DMA,
        pltpu.SemaphoreType.DMA,
        pltpu.SemaphoreType.DMA,
        pltpu.SemaphoreType.DMA,
        pltpu.SemaphoreType.DMA,
    ],
)
def _edge_alpha(xl_hbm, xr_hbm, src2_hbm, dst2_hbm, ew2_hbm, att_hbm, we_hbm,
                aexp2_hbm, denom_hbm,
                src2_v, dst2_v, ew2_v, ae2_v, ewb_v,
                xlr0_v, xrr0_v, xlr1_v, xrr1_v,
                st_v, att_v, we_v, z_v, denom_sp,
                semA0, semB0, semA1, semB1, semD):
    cid = lax.axis_index("c")
    sid = lax.axis_index("s")
    wid = sid * NC + cid
    rbase = wid * NCH

    zero16 = jnp.zeros((16,), jnp.float32)

    def _zb(i, _):
        z_v[pl.ds(i * 16, 16)] = zero16
        return 0

    lax.fori_loop(0, RPT // 16, _zb, 0)
    pltpu.sync_copy(z_v, denom_sp.at[pl.ds(sid * RPT, RPT)])

    pltpu.sync_copy(att_hbm, att_v)
    pltpu.sync_copy(we_hbm, we_v)
    pltpu.sync_copy(src2_hbm.at[pl.ds(rbase, NCH)], src2_v)
    pltpu.sync_copy(dst2_hbm.at[pl.ds(rbase, NCH)], dst2_v)
    pltpu.sync_copy(ew2_hbm.at[pl.ds(rbase, NCH)], ew2_v)
    plsc.subcore_barrier()

    attr = [att_v[pl.ds(16 * k, 16)] for k in range(H // 16)]
    wer = [we_v[pl.ds(16 * k, 16)] for k in range(H // 16)]
    bufs = ((xlr0_v, xrr0_v, semA0, semB0), (xlr1_v, xrr1_v, semA1, semB1))

    def _fire(c, b):
        xlr_b, xrr_b, semA, semB = bufs[b]
        pltpu.async_copy(xl_hbm.at[src2_v.at[c]], xlr_b, semA)
        pltpu.async_copy(xr_hbm.at[dst2_v.at[c]], xrr_b, semB)

    def _compute(c, b):
        xlr_b, xrr_b, semA, semB = bufs[b]
        pltpu.make_async_copy(xl_hbm.at[src2_v.at[c]], xlr_b, semA).wait()
        pltpu.make_async_copy(xr_hbm.at[dst2_v.at[c]], xrr_b, semB).wait()

        def grp_body(g, _):
            s = pl.ds(16 * g, 16)
            ewb_v[s] = ew2_v[c, s]
            for l in range(16):
                e = 16 * g + l
                ewes = plsc.load_gather(ewb_v, [jnp.full((16,), e, jnp.int32)])
                acc = jnp.zeros((16,), jnp.float32)
                for k in range(H // 16):
                    sk = pl.ds(16 * k, 16)
                    v = xlr_b[e, sk] + xrr_b[e, sk] + ewes * wer[k]
                    acc = acc + attr[k] * jnp.maximum(v, 0.2 * v)
                st_v[pl.ds(16 * l, 16)] = acc
            base16 = lax.iota(jnp.int32, 16) * 16
            tot = jnp.zeros((16,), jnp.float32)
            for i in range(16):
                tot = tot + plsc.load_gather(st_v, [base16 + i])
            ae2_v[c, s] = jnp.exp(tot)
            return 0

        lax.fori_loop(0, CH // 16, grp_body, 0)
        pltpu.async_copy(ae2_v.at[c], denom_sp.at[dst2_v.at[c]], semD,
                         add=True)

    _fire(0, 0)

    def pipe_body(t, _):
        c0 = 2 * t
        _fire(c0 + 1, 1)
        _compute(c0, 0)
        _fire(c0 + 2, 0)
        _compute(c0 + 1, 1)
        return 0

    lax.fori_loop(0, (NCH - 1) // 2, pipe_body, 0)
    _compute(NCH - 1, 0)

    pltpu.make_async_copy(ew2_hbm.at[pl.ds(rbase, NCH)], ae2_v, semD).wait()
    pltpu.sync_copy(ae2_v, aexp2_hbm.at[pl.ds(rbase, NCH)])

    plsc.subcore_barrier()
    pltpu.sync_copy(denom_sp.at[pl.ds(sid * RPT, RPT)],
                    denom_hbm.at[pl.ds(cid * NPAD + sid * RPT, RPT)])


@functools.partial(
    pl.kernel,
    mesh=_mesh,
    compiler_params=pltpu.CompilerParams(needs_layout_passes=False,
                                         use_tc_tiling_on_sc=False),
    out_type=jax.ShapeDtypeStruct((NC * NPAD, H), jnp.float32),
    scratch_types=[
        pltpu.VMEM((EPW,), jnp.int32),
        pltpu.VMEM((CH,), jnp.int32),
        pltpu.VMEM((CH,), jnp.float32),
        pltpu.VMEM((CH, H), jnp.float32),
        pltpu.VMEM((CH,), jnp.int32),
        pltpu.VMEM((CH,), jnp.float32),
        pltpu.VMEM((CH, H), jnp.float32),
        pltpu.VMEM((NPAD,), jnp.float32),
        pltpu.VMEM_SHARED((NPAD, H), jnp.float32),
        pltpu.SemaphoreType.DMA,
        pltpu.SemaphoreType.DMA,
        pltpu.SemaphoreType.DMA,
        pltpu.SemaphoreType.DMA,
        pltpu.SemaphoreType.DMA,
        pltpu.SemaphoreType.DMA,
    ],
)
def _edge_message(xl_hbm, src_hbm, dst_hbm, aexp_hbm, den_hbm, z_hbm,
                  outp_hbm,
                  src_v, dst0_v, ae0_v, xlr0_v, dst1_v, ae1_v, xlr1_v,
                  den_v, acc_sp,
                  semI0, semA0, semS0, semI1, semA1, semS1):
    cid = lax.axis_index("c")
    sid = lax.axis_index("s")
    wid = sid * NC + cid
    ebase = wid * EPW

    pltpu.sync_copy(z_hbm, acc_sp.at[pl.ds(sid * RPT, RPT)])
    pltpu.sync_copy(den_hbm, den_v)
    pltpu.sync_copy(src_hbm.at[pl.ds(ebase, EPW)], src_v)
    plsc.subcore_barrier()

    bufs = ((dst0_v, ae0_v, xlr0_v, semI0, semA0, semS0),
            (dst1_v, ae1_v, xlr1_v, semI1, semA1, semS1))

    def _fire(c, b, wait_scatter):
        dst_b, ae_b, xlr_b, semI, semA, semS = bufs[b]
        if wait_scatter:
            pltpu.make_async_copy(z_hbm.at[pl.ds(0, CH)], xlr_b, semS).wait()
        base = ebase + c * CH
        pltpu.async_copy(dst_hbm.at[pl.ds(base, CH)], dst_b, semI)
        pltpu.async_copy(aexp_hbm.at[pl.ds(base, CH)], ae_b, semI)
        pltpu.async_copy(xl_hbm.at[src_v.at[pl.ds(c * CH, CH)]], xlr_b, semA)

    def _compute(c, b):
        dst_b, ae_b, xlr_b, semI, semA, semS = bufs[b]
        pltpu.make_async_copy(dst_hbm.at[pl.ds(0, CH)], dst_b, semI).wait()
        pltpu.make_async_copy(aexp_hbm.at[pl.ds(0, CH)], ae_b, semI).wait()
        pltpu.make_async_copy(z_hbm.at[pl.ds(0, CH)], xlr_b, semA).wait()

        def grp_body(g, _):
            s = pl.ds(16 * g, 16)
            dst16 = dst_b[s]
            den = plsc.load_gather(den_v, [dst16]) + 1e-16
            ae_b[s] = ae_b[s] / den
            for l in range(16):
                e = 16 * g + l
                asp = plsc.load_gather(ae_b, [jnp.full((16,), e, jnp.int32)])
                for k in range(H // 16):
                    sk = pl.ds(16 * k, 16)
                    xlr_b[e, sk] = xlr_b[e, sk] * asp
            return 0

        lax.fori_loop(0, CH // 16, grp_body, 0)
        pltpu.async_copy(xlr_b, acc_sp.at[dst_b], semS, add=True)

    def pipe_body(t, _):
        c0 = 2 * t
        _fire(c0 + 1, 1, True)
        _compute(c0, 0)
        _fire(c0 + 2, 0, True)
        _compute(c0 + 1, 1)
        return 0

    _fire(0, 0, False)
    _fire(1, 1, False)
    _compute(0, 0)
    _fire(2, 0, True)
    _compute(1, 1)
    lax.fori_loop(1, (NCH - 1) // 2, pipe_body, 0)
    _compute(NCH - 1, 0)

    pltpu.make_async_copy(z_hbm.at[pl.ds(0, CH)], xlr1_v, semS1).wait()
    pltpu.make_async_copy(z_hbm.at[pl.ds(0, CH)], xlr0_v, semS0).wait()

    plsc.subcore_barrier()
    pltpu.sync_copy(acc_sp.at[pl.ds(sid * RPT, RPT)],
                    outp_hbm.at[pl.ds(cid * NPAD + sid * RPT, RPT)])


BR = 400


def _enc_body(x_ref, w_ref, b_ref, o_ref):
    o_ref[:] = jnp.dot(x_ref[:], w_ref[:],
                       preferred_element_type=jnp.float32) + b_ref[:]


def _encode(x, W_enc, b_enc):
    return pl.pallas_call(
        _enc_body,
        grid=(N // BR,),
        in_specs=[
            pl.BlockSpec((BR, H), lambda i: (i, 0)),
            pl.BlockSpec((H, H), lambda i: (0, 0)),
            pl.BlockSpec((1, H), lambda i: (0, 0)),
        ],
        out_specs=pl.BlockSpec((BR, H), lambda i: (i, 0)),
        out_shape=jax.ShapeDtypeStruct((N, H), jnp.float32),
    )(x, W_enc, b_enc.reshape(1, H))


def _dsum_body(d_ref, o_ref):
    o_ref[:] = d_ref[0:1, :] + d_ref[1:2, :]


def _denom_sum(denom):
    return pl.pallas_call(
        _dsum_body,
        out_shape=jax.ShapeDtypeStruct((1, NPAD), jnp.float32),
    )(denom.reshape(NC, NPAD)).reshape(NPAD)


def _ln_relu(h, g, b):
    mu = jnp.mean(h, axis=1, keepdims=True)
    var = jnp.mean((h - mu) ** 2, axis=1, keepdims=True)
    return jnp.maximum(g * (h - mu) * lax.rsqrt(var + 1e-5) + b, 0.0)


def _dense_body(h_ref, r0_ref, r1_ref, badd_ref, g_ref, b_ref,
                wl_ref, bl_ref, wr_ref, br_ref,
                hn_ref, xl_ref, xr_ref):
    hv = h_ref[:] + r0_ref[:] + r1_ref[:] + badd_ref[:]
    hn_ref[:] = hv
    a = _ln_relu(hv, g_ref[:], b_ref[:])
    xl_ref[:] = jnp.dot(a, wl_ref[:], preferred_element_type=jnp.float32) + bl_ref[:]
    xr_ref[:] = jnp.dot(a, wr_ref[:], preferred_element_type=jnp.float32) + br_ref[:]


def _dense_layer(h, r0, r1, badd, g, b, Wl, bl, Wr, br):
    vec = pl.BlockSpec((1, H), lambda i: (0, 0))
    mat = pl.BlockSpec((H, H), lambda i: (0, 0))
    row = pl.BlockSpec((BR, H), lambda i: (i, 0))
    return pl.pallas_call(
        _dense_body,
        grid=(N // BR,),
        in_specs=[row, row, row, vec, vec, vec, mat, vec, mat, vec],
        out_specs=(row, row, row),
        out_shape=(jax.ShapeDtypeStruct((N, H), jnp.float32),
                   jax.ShapeDtypeStruct((N, H), jnp.float32),
                   jax.ShapeDtypeStruct((N, H), jnp.float32)),
    )(h, r0, r1, badd.reshape(1, H), g.reshape(1, H), b.reshape(1, H),
      Wl, bl.reshape(1, H), Wr, br.reshape(1, H))


def _final_body(h_ref, r0_ref, r1_ref, badd_ref, g_ref, b_ref,
                wf_ref, bf_ref, y_ref):
    hv = h_ref[:] + r0_ref[:] + r1_ref[:] + badd_ref[:]
    a = _ln_relu(hv, g_ref[:], b_ref[:])
    y_ref[:] = jnp.dot(a, wf_ref[:], preferred_element_type=jnp.float32) + bf_ref[:]


def _final_layer(h, r0, r1, badd, g, b, W_fc, b_fc):
    vec = pl.BlockSpec((1, H), lambda i: (0, 0))
    row = pl.BlockSpec((BR, H), lambda i: (i, 0))
    return pl.pallas_call(
        _final_body,
        grid=(N // BR,),
        in_specs=[row, row, row, vec, vec, vec,
                  pl.BlockSpec((H, 1), lambda i: (0, 0)),
                  pl.BlockSpec((1, 1), lambda i: (0, 0))],
        out_specs=pl.BlockSpec((BR, 1), lambda i: (i, 0)),
        out_shape=jax.ShapeDtypeStruct((N, 1), jnp.float32),
    )(h, r0, r1, badd.reshape(1, H), g.reshape(1, H), b.reshape(1, H),
      W_fc, b_fc.reshape(1, 1))


def kernel(x, edge_index, edge_weight, W_enc, b_enc, ln_gamma, ln_beta,
           W_l, b_l, W_r, b_r, W_e, att, bias_out, ln_f_gamma, ln_f_beta,
           W_fc, b_fc):
    srcf = edge_index[0].astype(jnp.int32)
    dstf = edge_index[1].astype(jnp.int32)
    src2 = srcf.reshape(ECH, CH)
    dst2 = dstf.reshape(ECH, CH)
    ew2 = edge_weight[:, 0].reshape(ECH, CH)
    zrow = jnp.zeros((RPT, H), jnp.float32)
    zres = jnp.zeros((N, H), jnp.float32)

    h = _encode(x, W_enc, b_enc)
    r0 = zres
    r1 = zres
    badd = jnp.zeros((H,), jnp.float32)
    for i in range(L):
        h, xl, xr = _dense_layer(h, r0, r1, badd, ln_gamma[i], ln_beta[i],
                                 W_l[i], b_l[i], W_r[i], b_r[i])
        aexp2, denom = _edge_alpha(xl, xr, src2, dst2, ew2, att[i], W_e[i][0])
        outp = _edge_message(xl, srcf, dstf, aexp2.reshape(E),
                             _denom_sum(denom), zrow)
        r0 = outp[:N]
        r1 = outp[NPAD:NPAD + N]
        badd = bias_out[i]
    return _final_layer(h, r0, r1, badd, ln_f_gamma, ln_f_beta, W_fc, b_fc)
